# baseline (speedup 1.0000x reference)
import functools

import jax
import jax.numpy as jnp
from jax import lax
from jax.experimental import pallas as pl
from jax.experimental.pallas import tpu as pltpu
from jax.experimental.pallas import tpu_sc as plsc

DIM = 32
G = 4


def _make_gather(BATCH: int, HIST: int):
    info = plsc.get_sparse_core_info()
    NC, NS = info.num_cores, info.num_subcores
    NW = NC * NS
    assert BATCH % NW == 0
    rows_w = BATCH // NW
    assert rows_w % (2 * G) == 0
    n_groups = rows_w // G
    toks_w = rows_w * HIST
    splits = []
    off = 0
    while off < HIST:
        ln = min(128, HIST - off)
        splits.append((off, ln))
        off += ln
    assert all(o % 8 == 0 for o, _ in splits) and HIST % 8 == 0

    mesh = plsc.VectorSubcoreMesh(core_axis_name="c", subcore_axis_name="s")

    @functools.partial(
        pl.kernel,
        out_type=jax.ShapeDtypeStruct((BATCH, HIST, DIM), jnp.float32),
        mesh=mesh,
        scratch_types=[
            pltpu.VMEM((toks_w,), jnp.int32),
            pltpu.VMEM((G, HIST, DIM), jnp.float32),
            pltpu.VMEM((G, HIST, DIM), jnp.float32),
            pltpu.SemaphoreType.DMA,
            pltpu.SemaphoreType.DMA,
        ],
        compiler_params=pltpu.CompilerParams(use_tc_tiling_on_sc=False),
    )
    def emb(idx_hbm, table_hbm, out_hbm, idx_v, buf0, buf1, sem0, sem1):
        wid = lax.axis_index("s") * NC + lax.axis_index("c")
        pltpu.sync_copy(idx_hbm.at[pl.ds(wid * toks_w, toks_w)], idx_v)

        def fire(grp, buf, sem):
            for r in range(G):
                for off, ln in splits:
                    pltpu.async_copy(
                        table_hbm.at[idx_v.at[pl.ds((grp * G + r) * HIST + off, ln)]],
                        buf.at[r, pl.ds(off, ln)],
                        sem,
                    )

        def drain(buf, sem):
            for r in range(G):
                for off, ln in splits:
                    pltpu.make_async_copy(
                        table_hbm.at[idx_v.at[pl.ds(off, ln)]],
                        buf.at[r, pl.ds(off, ln)],
                        sem,
                    ).wait()

        def writeback(grp, buf):
            row0 = wid * rows_w + grp * G
            pltpu.sync_copy(buf, out_hbm.at[pl.ds(row0, G)])

        fire(0, buf0, sem0)

        def body(g, carry):
            fire(2 * g + 1, buf1, sem1)
            drain(buf0, sem0)
            writeback(2 * g, buf0)

            @pl.when(2 * g + 2 < n_groups)
            def _():
                fire(2 * g + 2, buf0, sem0)

            drain(buf1, sem1)
            writeback(2 * g + 1, buf1)
            return carry

        lax.fori_loop(0, n_groups // 2, body, 0)

    return emb


def kernel(token_ids, embed):
    BATCH, HIST = token_ids.shape
    idx = token_ids.reshape(-1).astype(jnp.int32)
    return _make_gather(BATCH, HIST)(idx, embed)
